# baseline (speedup 1.0000x reference)
import jax
import jax.numpy as jnp
from jax import lax
from jax.experimental import pallas as pl
from jax.experimental.pallas import tpu as pltpu
from jax.experimental.pallas import tpu_sc as plsc

N = 10000
E = 320000
D_IN = 128
H = 64
NUM_CLASSES = 2
EPS = 1e-5
BN_SCALE = float(1.0 / (1.0 + EPS) ** 0.5)

NC = 2
NS = 16
NW = NC * NS
EPT = E // NW
KE = 80
NB = EPT // KE
NP = 10240
RPT = NP // NS
ZR = 128

import functools


@functools.lru_cache(maxsize=None)
def _mesh():
    return plsc.VectorSubcoreMesh(
        core_axis_name="c", subcore_axis_name="s",
        num_cores=NC, num_subcores=NS)


def _deg_body(row_hbm, attr_hbm, out_hbm, row_v, attr_v, acc_v):
    cid = lax.axis_index("c")
    sid = lax.axis_index("s")
    wid = cid * NS + sid
    base = wid * EPT
    pltpu.sync_copy(row_hbm.at[pl.ds(base, EPT)], row_v)
    pltpu.sync_copy(attr_hbm.at[pl.ds(base, EPT)], attr_v)

    def zero(j, carry):
        acc_v[pl.ds(j * 16, 16)] = jnp.zeros((16,), jnp.float32)
        return carry
    lax.fori_loop(0, N // 16, zero, 0)

    def body(j, carry):
        idx = row_v[pl.ds(j * 16, 16)]
        a = jnp.abs(attr_v[pl.ds(j * 16, 16)])
        plsc.addupdate_scatter(acc_v, [idx], a)
        return carry
    lax.fori_loop(0, EPT // 16, body, 0)
    pltpu.sync_copy(acc_v, out_hbm.at[wid, 0])


@functools.lru_cache(maxsize=None)
def _deg_call():
    return pl.kernel(
        _deg_body,
        out_type=jax.ShapeDtypeStruct((NW, 1, N), jnp.float32),
        mesh=_mesh(),
        compiler_params=pltpu.CompilerParams(needs_layout_passes=False, use_tc_tiling_on_sc=False),
        scratch_types=[
            pltpu.VMEM((EPT,), jnp.int32),
            pltpu.VMEM((EPT,), jnp.float32),
            pltpu.VMEM((N,), jnp.float32),
        ],
    )


def _spmm_body(y_hbm, eb_hbm, out_hbm,
               ebig, rows0, rows1, zero_v, acc_sh,
               esem, gsem0, gsem1, ssem0, ssem1):
    cid = lax.axis_index("c")
    sid = lax.axis_index("s")
    wid = cid * NS + sid

    pltpu.async_copy(eb_hbm.at[wid], ebig, esem)

    def z(j, carry):
        for c in range(H // 16):
            zero_v[j, pl.ds(c * 16, 16)] = jnp.zeros((16,), jnp.float32)
        return carry
    lax.fori_loop(0, ZR, z, 0)
    for k in range(RPT // ZR):
        pltpu.sync_copy(zero_v, acc_sh.at[pl.ds(sid * RPT + k * ZR, ZR)])
    pltpu.make_async_copy(eb_hbm.at[wid], ebig, esem).wait()

    rows = (rows0, rows1)
    gsem = (gsem0, gsem1)
    ssem = (ssem0, ssem1)

    def gstart(b, s):
        pltpu.async_copy(y_hbm.at[ebig.at[3 * b]], rows[s], gsem[s])

    def gwait(b, s):
        pltpu.make_async_copy(y_hbm.at[ebig.at[3 * b]], rows[s], gsem[s]).wait()

    def sstart(b, s):
        pltpu.async_copy(rows[s], acc_sh.at[ebig.at[3 * b + 1]], ssem[s],
                         add=True)

    def sdrain(s):
        pltpu.make_async_copy(y_hbm.at[pl.ds(0, KE)], rows[s], ssem[s]).wait()

    def mul(b, s):
        def mul1(e, c2):
            w = plsc.load_gather(ebig.at[3 * b + 2],
                                 [jnp.full((16,), e, jnp.int32)])
            wf = plsc.bitcast(w, jnp.float32)
            for c in range(H // 16):
                rows[s][e, pl.ds(c * 16, 16)] = (
                    rows[s][e, pl.ds(c * 16, 16)] * wf)
            return c2
        lax.fori_loop(0, KE, mul1, 0, unroll=8)

    gstart(0, 0)
    plsc.subcore_barrier()

    gwait(0, 0)
    gstart(1, 1)
    mul(0, 0)
    sstart(0, 0)
    gwait(1, 1)
    sdrain(0)
    gstart(2, 0)
    mul(1, 1)
    sstart(1, 1)

    def pair(i, carry):
        b0 = 2 * i
        gwait(b0, 0)
        sdrain(1)
        gstart(b0 + 1, 1)
        mul(b0, 0)
        sstart(b0, 0)

        b1 = 2 * i + 1
        gwait(b1, 1)
        sdrain(0)
        gstart(b1 + 1, 0)
        mul(b1, 1)
        sstart(b1, 1)
        return carry
    lax.fori_loop(1, (NB - 1) // 2, pair, 0)

    gwait(NB - 1, 0)
    mul(NB - 1, 0)
    sstart(NB - 1, 0)
    sdrain(1)
    sdrain(0)
    plsc.subcore_barrier()
    pltpu.sync_copy(acc_sh.at[pl.ds(sid * RPT, RPT)],
                    out_hbm.at[cid, pl.ds(sid * RPT, RPT)])


@functools.lru_cache(maxsize=None)
def _spmm_call():
    return pl.kernel(
        _spmm_body,
        out_type=jax.ShapeDtypeStruct((NC, NP, H), jnp.float32),
        mesh=_mesh(),
        compiler_params=pltpu.CompilerParams(needs_layout_passes=False, use_tc_tiling_on_sc=False),
        scratch_types=[
            pltpu.VMEM((NB * 3, KE), jnp.int32),
            pltpu.VMEM((KE, H), jnp.float32),
            pltpu.VMEM((KE, H), jnp.float32),
            pltpu.VMEM((ZR, H), jnp.float32),
            pltpu.VMEM_SHARED((NP, H), jnp.float32),
            pltpu.SemaphoreType.DMA,
            pltpu.SemaphoreType.DMA,
            pltpu.SemaphoreType.DMA,
            pltpu.SemaphoreType.DMA,
            pltpu.SemaphoreType.DMA,
        ],
    )


_RB = 2000
_RBP = 2048
_GRID = (N // _RB,)


def _prep_body(degp_ref, x_ref, w_ref, o_dinv, o_y):
    ones = jnp.ones((NW, 1), jnp.float32)
    deg = lax.dot_general(degp_ref[...], ones, (((0,), (0,)), ((), ())),
                          preferred_element_type=jnp.float32,
                          precision=lax.Precision.HIGHEST)
    deg_safe = jnp.where(deg > 0, deg, 1.0)
    dinv = jnp.where(deg > 0, lax.rsqrt(deg_safe), 0.0)
    o_dinv[...] = jnp.concatenate(
        [dinv, jnp.zeros((NP - N, 1), jnp.float32)], axis=0)
    y = jnp.dot(x_ref[...], w_ref[...], preferred_element_type=jnp.float32)
    o_y[...] = jnp.concatenate(
        [y * dinv, jnp.zeros((NP - N, H), jnp.float32)], axis=0)


def _prep(degp, x, w):
    return pl.pallas_call(
        _prep_body,
        in_specs=[
            pl.BlockSpec((NW, N), lambda: (0, 0)),
            pl.BlockSpec((N, D_IN), lambda: (0, 0)),
            pl.BlockSpec((D_IN, H), lambda: (0, 0)),
        ],
        out_specs=[
            pl.BlockSpec((NP, 1), lambda: (0, 0)),
            pl.BlockSpec((NP, H), lambda: (0, 0)),
        ],
        out_shape=[
            jax.ShapeDtypeStruct((NP, 1), jnp.float32),
            jax.ShapeDtypeStruct((NP, H), jnp.float32),
        ],
    )(degp, x, w)


def _mid_body(p_ref, dinv_ref, b_ref, g_ref, be_ref, w_ref, o_ref):
    conv = (p_ref[0] + p_ref[1]) * dinv_ref[...] + b_ref[...]
    h = jnp.maximum(conv * (g_ref[...] * BN_SCALE) + be_ref[...], 0.0)
    y = jnp.dot(h, w_ref[...], preferred_element_type=jnp.float32)
    o_ref[...] = y * dinv_ref[...]


def _mid(p, dinv, b, g, be, w_next):
    return pl.pallas_call(
        _mid_body,
        grid=(NP // _RBP,),
        in_specs=[
            pl.BlockSpec((NC, _RBP, H), lambda i: (0, i, 0)),
            pl.BlockSpec((_RBP, 1), lambda i: (i, 0)),
            pl.BlockSpec((1, H), lambda i: (0, 0)),
            pl.BlockSpec((1, H), lambda i: (0, 0)),
            pl.BlockSpec((1, H), lambda i: (0, 0)),
            pl.BlockSpec((H, H), lambda i: (0, 0)),
        ],
        out_specs=pl.BlockSpec((_RBP, H), lambda i: (i, 0)),
        out_shape=jax.ShapeDtypeStruct((NP, H), jnp.float32),
    )(p, dinv, b, g, be, w_next)


def _head_body(p_ref, dinv_ref, b_ref, g_ref, be_ref,
               wc1_ref, bc1_ref, wc2_ref, bc2_ref, o_ref):
    conv = (p_ref[0] + p_ref[1]) * dinv_ref[...] + b_ref[...]
    h = jnp.maximum(conv * (g_ref[...] * BN_SCALE) + be_ref[...], 0.0)
    z = jnp.maximum(
        jnp.dot(h, wc1_ref[...], preferred_element_type=jnp.float32)
        + bc1_ref[...], 0.0)
    o_ref[...] = (jnp.dot(z, wc2_ref[...], preferred_element_type=jnp.float32)
                  + bc2_ref[...])


def _head(p, dinv, b, g, be, wc1, bc1, wc2, bc2):
    return pl.pallas_call(
        _head_body,
        grid=_GRID,
        in_specs=[
            pl.BlockSpec((NC, _RB, H), lambda i: (0, i, 0)),
            pl.BlockSpec((_RB, 1), lambda i: (i, 0)),
            pl.BlockSpec((1, H), lambda i: (0, 0)),
            pl.BlockSpec((1, H), lambda i: (0, 0)),
            pl.BlockSpec((1, H), lambda i: (0, 0)),
            pl.BlockSpec((H, H // 2), lambda i: (0, 0)),
            pl.BlockSpec((1, H // 2), lambda i: (0, 0)),
            pl.BlockSpec((H // 2, NUM_CLASSES), lambda i: (0, 0)),
            pl.BlockSpec((1, NUM_CLASSES), lambda i: (0, 0)),
        ],
        out_specs=pl.BlockSpec((_RB, NUM_CLASSES), lambda i: (i, 0)),
        out_shape=jax.ShapeDtypeStruct((N, NUM_CLASSES), jnp.float32),
    )(p, dinv, b, g, be, wc1, bc1, wc2, bc2)


def kernel(x, edge_index, edge_attr, W1, b1, W2, b2, W3, b3,
           g1, be1, g2, be2, g3, be3, Wc1, bc1, Wc2, bc2):
    row = edge_index[0]
    col = edge_index[1]
    attr = edge_attr[:, 0]
    eb = jnp.concatenate([
        row.reshape(NW, NB, 1, KE),
        col.reshape(NW, NB, 1, KE),
        lax.bitcast_convert_type(attr, jnp.int32).reshape(NW, NB, 1, KE),
    ], axis=2).reshape(NW, NB * 3, KE)

    degp = _deg_call()(row, attr)
    dinv, y = _prep(degp.reshape(NW, N), x, W1)
    p = _spmm_call()(y, eb)
    y = _mid(p, dinv, b1.reshape(1, H), g1.reshape(1, H), be1.reshape(1, H), W2)
    p = _spmm_call()(y, eb)
    y = _mid(p, dinv, b2.reshape(1, H), g2.reshape(1, H), be2.reshape(1, H), W3)
    p = _spmm_call()(y, eb)
    return _head(p, dinv, b3.reshape(1, H), g3.reshape(1, H), be3.reshape(1, H),
                 Wc1, bc1.reshape(1, H // 2), Wc2, bc2.reshape(1, NUM_CLASSES))


# speedup vs baseline: 1.0343x; 1.0343x over previous
import jax
import jax.numpy as jnp
from jax import lax
from jax.experimental import pallas as pl
from jax.experimental.pallas import tpu as pltpu
from jax.experimental.pallas import tpu_sc as plsc

N = 10000
E = 320000
D_IN = 128
H = 64
NUM_CLASSES = 2
EPS = 1e-5
BN_SCALE = float(1.0 / (1.0 + EPS) ** 0.5)

NC = 2
NS = 16
NW = NC * NS
EPT = E // NW
KE = 80
NB = EPT // KE
SBK = 5
NSB = NB // SBK
NP = 10240
RPT = NP // NS
ZR = 128

import functools


@functools.lru_cache(maxsize=None)
def _mesh():
    return plsc.VectorSubcoreMesh(
        core_axis_name="c", subcore_axis_name="s",
        num_cores=NC, num_subcores=NS)


def _deg_body(row_hbm, attr_hbm, out_hbm, row_v, attr_v, acc_v):
    cid = lax.axis_index("c")
    sid = lax.axis_index("s")
    wid = cid * NS + sid
    base = wid * EPT
    pltpu.sync_copy(row_hbm.at[pl.ds(base, EPT)], row_v)
    pltpu.sync_copy(attr_hbm.at[pl.ds(base, EPT)], attr_v)

    def zero(j, carry):
        acc_v[pl.ds(j * 16, 16)] = jnp.zeros((16,), jnp.float32)
        return carry
    lax.fori_loop(0, N // 16, zero, 0)

    def body(j, carry):
        idx = row_v[pl.ds(j * 16, 16)]
        a = jnp.abs(attr_v[pl.ds(j * 16, 16)])
        plsc.addupdate_scatter(acc_v, [idx], a)
        return carry
    lax.fori_loop(0, EPT // 16, body, 0)
    pltpu.sync_copy(acc_v, out_hbm.at[wid, 0])


@functools.lru_cache(maxsize=None)
def _deg_call():
    return pl.kernel(
        _deg_body,
        out_type=jax.ShapeDtypeStruct((NW, 1, N), jnp.float32),
        mesh=_mesh(),
        compiler_params=pltpu.CompilerParams(needs_layout_passes=False, use_tc_tiling_on_sc=False),
        scratch_types=[
            pltpu.VMEM((EPT,), jnp.int32),
            pltpu.VMEM((EPT,), jnp.float32),
            pltpu.VMEM((N,), jnp.float32),
        ],
    )


def _spmm_body(y_hbm, eb_hbm, out_hbm,
               ebig, rows0, rows1, zero_v, acc_sh,
               esem, gsem0, gsem1, ssem0, ssem1):
    cid = lax.axis_index("c")
    sid = lax.axis_index("s")
    wid = cid * NS + sid

    pltpu.async_copy(eb_hbm.at[wid], ebig, esem)

    def z(j, carry):
        for c in range(H // 16):
            zero_v[j, pl.ds(c * 16, 16)] = jnp.zeros((16,), jnp.float32)
        return carry
    lax.fori_loop(0, ZR, z, 0)
    for k in range(RPT // ZR):
        pltpu.sync_copy(zero_v, acc_sh.at[pl.ds(sid * RPT + k * ZR, ZR)])
    pltpu.make_async_copy(eb_hbm.at[wid], ebig, esem).wait()

    rows = (rows0, rows1)
    gsem = (gsem0, gsem1)
    ssem = (ssem0, ssem1)

    def gstart(sb, s):
        for j in range(SBK):
            pltpu.async_copy(y_hbm.at[ebig.at[3 * (SBK * sb + j)]],
                             rows[s].at[pl.ds(KE * j, KE)], gsem[s])

    def gwait(sb, s):
        for j in range(SBK):
            pltpu.make_async_copy(y_hbm.at[ebig.at[3 * (SBK * sb + j)]],
                                  rows[s].at[pl.ds(KE * j, KE)],
                                  gsem[s]).wait()

    def sstart(sb, s):
        for j in range(SBK):
            pltpu.async_copy(rows[s].at[pl.ds(KE * j, KE)],
                             acc_sh.at[ebig.at[3 * (SBK * sb + j) + 1]],
                             ssem[s], add=True)

    def sdrain(s):
        for j in range(SBK):
            pltpu.make_async_copy(y_hbm.at[pl.ds(0, KE)],
                                  rows[s].at[pl.ds(KE * j, KE)],
                                  ssem[s]).wait()

    def mul(sb, s):
        for j in range(SBK):
            def mul1(e, c2, _j=j):
                w = plsc.load_gather(ebig.at[3 * (SBK * sb + _j) + 2],
                                     [jnp.full((16,), e, jnp.int32)])
                wf = plsc.bitcast(w, jnp.float32)
                for c in range(H // 16):
                    rows[s][KE * _j + e, pl.ds(c * 16, 16)] = (
                        rows[s][KE * _j + e, pl.ds(c * 16, 16)] * wf)
                return c2
            lax.fori_loop(0, KE, mul1, 0, unroll=8)

    gstart(0, 0)
    plsc.subcore_barrier()

    gwait(0, 0)
    gstart(1, 1)
    mul(0, 0)
    sstart(0, 0)

    def pair(i, carry):
        sb0 = 2 * i + 1
        gwait(sb0, 1)
        sdrain(0)
        gstart(sb0 + 1, 0)
        mul(sb0, 1)
        sstart(sb0, 1)

        sb1 = 2 * i + 2
        gwait(sb1, 0)
        @pl.when(sb1 + 1 < NSB)
        def _():
            sdrain(1)
            gstart(sb1 + 1, 1)
        mul(sb1, 0)
        sstart(sb1, 0)
        return carry
    lax.fori_loop(0, (NSB - 1) // 2, pair, 0)

    sdrain(1)
    sdrain(0)
    plsc.subcore_barrier()
    pltpu.sync_copy(acc_sh.at[pl.ds(sid * RPT, RPT)],
                    out_hbm.at[cid, pl.ds(sid * RPT, RPT)])


@functools.lru_cache(maxsize=None)
def _spmm_call():
    return pl.kernel(
        _spmm_body,
        out_type=jax.ShapeDtypeStruct((NC, NP, H), jnp.float32),
        mesh=_mesh(),
        compiler_params=pltpu.CompilerParams(needs_layout_passes=False, use_tc_tiling_on_sc=False),
        scratch_types=[
            pltpu.VMEM((NB * 3, KE), jnp.int32),
            pltpu.VMEM((SBK * KE, H), jnp.float32),
            pltpu.VMEM((SBK * KE, H), jnp.float32),
            pltpu.VMEM((ZR, H), jnp.float32),
            pltpu.VMEM_SHARED((NP, H), jnp.float32),
            pltpu.SemaphoreType.DMA,
            pltpu.SemaphoreType.DMA,
            pltpu.SemaphoreType.DMA,
            pltpu.SemaphoreType.DMA,
            pltpu.SemaphoreType.DMA,
        ],
    )


_RB = 2000
_RBP = 2048
_GRID = (N // _RB,)


def _prep_body(degp_ref, x_ref, w_ref, o_dinv, o_y):
    ones = jnp.ones((NW, 1), jnp.float32)
    deg = lax.dot_general(degp_ref[...], ones, (((0,), (0,)), ((), ())),
                          preferred_element_type=jnp.float32,
                          precision=lax.Precision.HIGHEST)
    deg_safe = jnp.where(deg > 0, deg, 1.0)
    dinv = jnp.where(deg > 0, lax.rsqrt(deg_safe), 0.0)
    o_dinv[...] = jnp.concatenate(
        [dinv, jnp.zeros((NP - N, 1), jnp.float32)], axis=0)
    y = jnp.dot(x_ref[...], w_ref[...], preferred_element_type=jnp.float32)
    o_y[...] = jnp.concatenate(
        [y * dinv, jnp.zeros((NP - N, H), jnp.float32)], axis=0)


def _prep(degp, x, w):
    return pl.pallas_call(
        _prep_body,
        in_specs=[
            pl.BlockSpec((NW, N), lambda: (0, 0)),
            pl.BlockSpec((N, D_IN), lambda: (0, 0)),
            pl.BlockSpec((D_IN, H), lambda: (0, 0)),
        ],
        out_specs=[
            pl.BlockSpec((NP, 1), lambda: (0, 0)),
            pl.BlockSpec((NP, H), lambda: (0, 0)),
        ],
        out_shape=[
            jax.ShapeDtypeStruct((NP, 1), jnp.float32),
            jax.ShapeDtypeStruct((NP, H), jnp.float32),
        ],
    )(degp, x, w)


def _mid_body(p_ref, dinv_ref, b_ref, g_ref, be_ref, w_ref, o_ref):
    conv = (p_ref[0] + p_ref[1]) * dinv_ref[...] + b_ref[...]
    h = jnp.maximum(conv * (g_ref[...] * BN_SCALE) + be_ref[...], 0.0)
    y = jnp.dot(h, w_ref[...], preferred_element_type=jnp.float32)
    o_ref[...] = y * dinv_ref[...]


def _mid(p, dinv, b, g, be, w_next):
    return pl.pallas_call(
        _mid_body,
        grid=(NP // _RBP,),
        in_specs=[
            pl.BlockSpec((NC, _RBP, H), lambda i: (0, i, 0)),
            pl.BlockSpec((_RBP, 1), lambda i: (i, 0)),
            pl.BlockSpec((1, H), lambda i: (0, 0)),
            pl.BlockSpec((1, H), lambda i: (0, 0)),
            pl.BlockSpec((1, H), lambda i: (0, 0)),
            pl.BlockSpec((H, H), lambda i: (0, 0)),
        ],
        out_specs=pl.BlockSpec((_RBP, H), lambda i: (i, 0)),
        out_shape=jax.ShapeDtypeStruct((NP, H), jnp.float32),
    )(p, dinv, b, g, be, w_next)


def _head_body(p_ref, dinv_ref, b_ref, g_ref, be_ref,
               wc1_ref, bc1_ref, wc2_ref, bc2_ref, o_ref):
    conv = (p_ref[0] + p_ref[1]) * dinv_ref[...] + b_ref[...]
    h = jnp.maximum(conv * (g_ref[...] * BN_SCALE) + be_ref[...], 0.0)
    z = jnp.maximum(
        jnp.dot(h, wc1_ref[...], preferred_element_type=jnp.float32)
        + bc1_ref[...], 0.0)
    o_ref[...] = (jnp.dot(z, wc2_ref[...], preferred_element_type=jnp.float32)
                  + bc2_ref[...])


def _head(p, dinv, b, g, be, wc1, bc1, wc2, bc2):
    return pl.pallas_call(
        _head_body,
        grid=_GRID,
        in_specs=[
            pl.BlockSpec((NC, _RB, H), lambda i: (0, i, 0)),
            pl.BlockSpec((_RB, 1), lambda i: (i, 0)),
            pl.BlockSpec((1, H), lambda i: (0, 0)),
            pl.BlockSpec((1, H), lambda i: (0, 0)),
            pl.BlockSpec((1, H), lambda i: (0, 0)),
            pl.BlockSpec((H, H // 2), lambda i: (0, 0)),
            pl.BlockSpec((1, H // 2), lambda i: (0, 0)),
            pl.BlockSpec((H // 2, NUM_CLASSES), lambda i: (0, 0)),
            pl.BlockSpec((1, NUM_CLASSES), lambda i: (0, 0)),
        ],
        out_specs=pl.BlockSpec((_RB, NUM_CLASSES), lambda i: (i, 0)),
        out_shape=jax.ShapeDtypeStruct((N, NUM_CLASSES), jnp.float32),
    )(p, dinv, b, g, be, wc1, bc1, wc2, bc2)


def kernel(x, edge_index, edge_attr, W1, b1, W2, b2, W3, b3,
           g1, be1, g2, be2, g3, be3, Wc1, bc1, Wc2, bc2):
    row = edge_index[0]
    col = edge_index[1]
    attr = edge_attr[:, 0]
    eb = jnp.concatenate([
        row.reshape(NW, NB, 1, KE),
        col.reshape(NW, NB, 1, KE),
        lax.bitcast_convert_type(attr, jnp.int32).reshape(NW, NB, 1, KE),
    ], axis=2).reshape(NW, NB * 3, KE)

    degp = _deg_call()(row, attr)
    dinv, y = _prep(degp.reshape(NW, N), x, W1)
    p = _spmm_call()(y, eb)
    y = _mid(p, dinv, b1.reshape(1, H), g1.reshape(1, H), be1.reshape(1, H), W2)
    p = _spmm_call()(y, eb)
    y = _mid(p, dinv, b2.reshape(1, H), g2.reshape(1, H), be2.reshape(1, H), W3)
    p = _spmm_call()(y, eb)
    return _head(p, dinv, b3.reshape(1, H), g3.reshape(1, H), be3.reshape(1, H),
                 Wc1, bc1.reshape(1, H // 2), Wc2, bc2.reshape(1, NUM_CLASSES))
